# baseline (speedup 1.0000x reference)
import functools

import jax
import jax.numpy as jnp
from jax import lax
from jax.experimental import pallas as pl
from jax.experimental.pallas import tpu as pltpu
from jax.experimental.pallas import tpu_sc as plsc

N_COEF = 0.3

B, D = 4096, 2048
NC, NS, L = 2, 16, 16
NW = NC * NS
ROWS_PER_W = B // NW
CHUNK = 16
NCHUNK = ROWS_PER_W // CHUNK
VECS = CHUNK * D // L


def _sc_body(x_hbm, r_hbm, out_hbm, idx_v, gbuf, xbuf, gsem, xsem, osem):
    wid = lax.axis_index("s") * NC + lax.axis_index("c")
    base = wid * ROWS_PER_W

    pltpu.sync_copy(r_hbm.at[wid], idx_v)

    for c in range(NCHUNK):
        row0 = base + c * CHUNK
        g_cp = pltpu.async_copy(x_hbm.at[idx_v.at[c]], gbuf, gsem)
        x_cp = pltpu.async_copy(x_hbm.at[pl.ds(row0, CHUNK)], xbuf, xsem)
        g_cp.wait()
        x_cp.wait()

        def blend(k, _):
            i = k // (D // L)
            j = (k % (D // L)) * L
            g = gbuf[i, pl.ds(j, L)]
            xv = xbuf[i, pl.ds(j, L)]
            gbuf[i, pl.ds(j, L)] = xv + N_COEF * (g - xv)
            return _

        lax.fori_loop(0, VECS, blend, 0, unroll=4)
        pltpu.async_copy(gbuf, out_hbm.at[pl.ds(row0, CHUNK)], osem).wait()


@jax.jit
def _batchout(x, r3):
    mesh = plsc.VectorSubcoreMesh(core_axis_name="c", subcore_axis_name="s")
    run = pl.kernel(
        _sc_body,
        out_type=jax.ShapeDtypeStruct((B, D), jnp.float32),
        mesh=mesh,
        scratch_types=[
            pltpu.VMEM((NCHUNK, CHUNK), jnp.int32),
            pltpu.VMEM((CHUNK, D), jnp.float32),
            pltpu.VMEM((CHUNK, D), jnp.float32),
            pltpu.SemaphoreType.DMA,
            pltpu.SemaphoreType.DMA,
            pltpu.SemaphoreType.DMA,
        ],
    )
    return run(x, r3)


def kernel(x, y, r):
    x_out = _batchout(x, r.reshape(NW, NCHUNK, CHUNK))
    return (x_out, r)


# speedup vs baseline: 1.4654x; 1.4654x over previous
import jax
import jax.numpy as jnp
from jax import lax
from jax.experimental import pallas as pl
from jax.experimental.pallas import tpu as pltpu
from jax.experimental.pallas import tpu_sc as plsc

N_COEF = 0.3

B, D = 4096, 2048
NC, NS, L = 2, 16, 16
NW = NC * NS
ROWS_PER_W = B // NW
CHUNK = 8
NCHUNK = ROWS_PER_W // CHUNK
VECS = CHUNK * D // L
JSHIFT = (D // L).bit_length() - 1


def _sc_body(x_hbm, r_hbm, out_hbm, idx_v, gbuf, xbuf, obuf, sems):
    wid = lax.axis_index("s") * NC + lax.axis_index("c")
    base = wid * ROWS_PER_W

    pltpu.sync_copy(r_hbm.at[wid], idx_v)

    def issue_in(c):
        s = c & 1
        g = pltpu.async_copy(x_hbm.at[idx_v.at[c]], gbuf.at[s], sems.at[s])
        x = pltpu.async_copy(
            x_hbm.at[pl.ds(base + c * CHUNK, CHUNK)], xbuf.at[s], sems.at[2 + s])
        return g, x

    in_flight = {0: issue_in(0)}
    out_flight = {}
    for c in range(NCHUNK):
        s = c & 1
        if c + 1 < NCHUNK:
            in_flight[c + 1] = issue_in(c + 1)
        g_cp, x_cp = in_flight.pop(c)
        g_cp.wait()
        x_cp.wait()
        if c - 2 in out_flight:
            out_flight.pop(c - 2).wait()

        def blend(k, _):
            i = k >> JSHIFT
            j = (k - (i << JSHIFT)) * L
            g = gbuf[s, i, pl.ds(j, L)]
            xv = xbuf[s, i, pl.ds(j, L)]
            obuf[s, i, pl.ds(j, L)] = xv + N_COEF * (g - xv)
            return _

        lax.fori_loop(0, VECS, blend, 0, unroll=8)
        out_flight[c] = pltpu.async_copy(
            obuf.at[s], out_hbm.at[pl.ds(base + c * CHUNK, CHUNK)], sems.at[4 + s])
    for cp in out_flight.values():
        cp.wait()


@jax.jit
def _batchout(x, r3):
    mesh = plsc.VectorSubcoreMesh(core_axis_name="c", subcore_axis_name="s")
    run = pl.kernel(
        _sc_body,
        out_type=jax.ShapeDtypeStruct((B, D), jnp.float32),
        mesh=mesh,
        scratch_types=[
            pltpu.VMEM((NCHUNK, CHUNK), jnp.int32),
            pltpu.VMEM((2, CHUNK, D), jnp.float32),
            pltpu.VMEM((2, CHUNK, D), jnp.float32),
            pltpu.VMEM((2, CHUNK, D), jnp.float32),
            pltpu.SemaphoreType.DMA((6,)),
        ],
    )
    return run(x, r3)


def kernel(x, y, r):
    x_out = _batchout(x, r.reshape(NW, NCHUNK, CHUNK))
    return (x_out, r)


# speedup vs baseline: 1.4659x; 1.0003x over previous
import jax
import jax.numpy as jnp
from jax import lax
from jax.experimental import pallas as pl
from jax.experimental.pallas import tpu as pltpu
from jax.experimental.pallas import tpu_sc as plsc

N_COEF = 0.3

B, D = 4096, 2048
NC, NS, L = 2, 16, 16
NW = NC * NS
ROWS_PER_W = B // NW
CHUNK = 8
NCHUNK = ROWS_PER_W // CHUNK
VECS = CHUNK * D // L
JSHIFT = (D // L).bit_length() - 1


def _sc_body(x_hbm, r_hbm, out_hbm, idx_v, gbuf, xbuf, obuf, sems):
    wid = lax.axis_index("s") * NC + lax.axis_index("c")
    base = wid * ROWS_PER_W

    pltpu.sync_copy(r_hbm.at[pl.ds(base, ROWS_PER_W)], idx_v)

    def issue_in(c):
        s = c & 1
        g = pltpu.async_copy(
            x_hbm.at[idx_v.at[pl.ds(c * CHUNK, CHUNK)]], gbuf.at[s], sems.at[s])
        x = pltpu.async_copy(
            x_hbm.at[pl.ds(base + c * CHUNK, CHUNK)], xbuf.at[s], sems.at[2 + s])
        return g, x

    in_flight = {0: issue_in(0)}
    out_flight = {}
    for c in range(NCHUNK):
        s = c & 1
        if c + 1 < NCHUNK:
            in_flight[c + 1] = issue_in(c + 1)
        g_cp, x_cp = in_flight.pop(c)
        g_cp.wait()
        x_cp.wait()
        if c - 2 in out_flight:
            out_flight.pop(c - 2).wait()

        def blend(k, _):
            i = k >> JSHIFT
            j = (k - (i << JSHIFT)) * L
            g = gbuf[s, i, pl.ds(j, L)]
            xv = xbuf[s, i, pl.ds(j, L)]
            obuf[s, i, pl.ds(j, L)] = xv + N_COEF * (g - xv)
            return _

        lax.fori_loop(0, VECS, blend, 0, unroll=8)
        out_flight[c] = pltpu.async_copy(
            obuf.at[s], out_hbm.at[pl.ds(base + c * CHUNK, CHUNK)], sems.at[4 + s])
    for cp in out_flight.values():
        cp.wait()


@jax.jit
def _batchout(x, r3):
    mesh = plsc.VectorSubcoreMesh(core_axis_name="c", subcore_axis_name="s")
    run = pl.kernel(
        _sc_body,
        out_type=jax.ShapeDtypeStruct((B, D), jnp.float32),
        mesh=mesh,
        scratch_types=[
            pltpu.VMEM((ROWS_PER_W,), jnp.int32),
            pltpu.VMEM((2, CHUNK, D), jnp.float32),
            pltpu.VMEM((2, CHUNK, D), jnp.float32),
            pltpu.VMEM((2, CHUNK, D), jnp.float32),
            pltpu.SemaphoreType.DMA((6,)),
        ],
    )
    return run(x, r3)


def kernel(x, y, r):
    x_out = _batchout(x, r)
    return (x_out, r)


# speedup vs baseline: 1.5228x; 1.0388x over previous
import jax
import jax.numpy as jnp
from jax import lax
from jax.experimental import pallas as pl
from jax.experimental.pallas import tpu as pltpu
from jax.experimental.pallas import tpu_sc as plsc

N_COEF = 0.3

B, D = 4096, 2048
NC, NS, L = 2, 16, 16
NW = NC * NS
ROWS_PER_W = B // NW
CHUNK = 8
NCHUNK = ROWS_PER_W // CHUNK
VECS = CHUNK * D // L
JSHIFT = (D // L).bit_length() - 1


def _sc_body(x_hbm, r_hbm, out_hbm, idx_v, gbuf, xbuf, obuf, sems):
    wid = lax.axis_index("s") * NC + lax.axis_index("c")
    base = wid * ROWS_PER_W

    pltpu.sync_copy(r_hbm.at[pl.ds(base, ROWS_PER_W)], idx_v)

    def issue_in(c, s):
        pltpu.async_copy(
            x_hbm.at[idx_v.at[pl.ds(c * CHUNK, CHUNK)]], gbuf.at[s], sems.at[s])
        pltpu.async_copy(
            x_hbm.at[pl.ds(base + c * CHUNK, CHUNK)], xbuf.at[s], sems.at[2 + s])

    def wait_in(s):
        pltpu.make_async_copy(x_hbm.at[pl.ds(0, CHUNK)], gbuf.at[s],
                              sems.at[s]).wait()
        pltpu.make_async_copy(x_hbm.at[pl.ds(0, CHUNK)], xbuf.at[s],
                              sems.at[2 + s]).wait()

    def issue_out(c, s):
        pltpu.async_copy(
            obuf.at[s], out_hbm.at[pl.ds(base + c * CHUNK, CHUNK)],
            sems.at[4 + s])

    def wait_out(s):
        pltpu.make_async_copy(obuf.at[s], out_hbm.at[pl.ds(0, CHUNK)],
                              sems.at[4 + s]).wait()

    issue_in(0, 0)
    issue_in(1, 1)

    def step(c, carry):
        s = c & 1
        wait_in(s)

        @pl.when(c >= 2)
        def _drain():
            wait_out(s)

        def blend(k, _):
            i = k >> JSHIFT
            j = (k - (i << JSHIFT)) * L
            g = gbuf[s, i, pl.ds(j, L)]
            xv = xbuf[s, i, pl.ds(j, L)]
            obuf[s, i, pl.ds(j, L)] = xv + N_COEF * (g - xv)
            return _

        lax.fori_loop(0, VECS, blend, 0, unroll=8)
        issue_out(c, s)

        @pl.when(c + 2 < NCHUNK)
        def _prefetch():
            issue_in(c + 2, s)

        return carry

    lax.fori_loop(0, NCHUNK, step, 0)
    wait_out(NCHUNK & 1)
    wait_out((NCHUNK + 1) & 1)


@jax.jit
def _batchout(x, r):
    mesh = plsc.VectorSubcoreMesh(core_axis_name="c", subcore_axis_name="s")
    run = pl.kernel(
        _sc_body,
        out_type=jax.ShapeDtypeStruct((B, D), jnp.float32),
        mesh=mesh,
        scratch_types=[
            pltpu.VMEM((ROWS_PER_W,), jnp.int32),
            pltpu.VMEM((2, CHUNK, D), jnp.float32),
            pltpu.VMEM((2, CHUNK, D), jnp.float32),
            pltpu.VMEM((2, CHUNK, D), jnp.float32),
            pltpu.SemaphoreType.DMA((6,)),
        ],
    )
    return run(x, r)


def kernel(x, y, r):
    x_out = _batchout(x, r)
    return (x_out, r)
